# baseline (speedup 1.0000x reference)
import functools

import jax
import jax.numpy as jnp
from jax import lax
from jax.experimental import pallas as pl
from jax.experimental.pallas import tpu as pltpu, tpu_sc as plsc

NW = 32
STEP_E = 32
NBUF = 2


def _ceil_to(x, m):
    return (x + m - 1) // m * m


def _sc_gather(E_in, C, E_pad, E_up):
    S = E_pad // (NW * STEP_E)
    assert S * NW * STEP_E == E_pad and S % NBUF == 0
    n_chunks = E_pad // STEP_E

    mesh = plsc.VectorSubcoreMesh(core_axis_name="c", subcore_axis_name="s")
    scratch = [
        pltpu.VMEM((NBUF, 5, STEP_E), jnp.int32),
        pltpu.VMEM((NBUF, 5, STEP_E), jnp.int32),
        [pltpu.VMEM((5, STEP_E, C), jnp.float32) for _ in range(NBUF)],
        [pltpu.SemaphoreType.DMA for _ in range(NBUF)],
        [pltpu.SemaphoreType.DMA for _ in range(NBUF)],
        [pltpu.SemaphoreType.DMA for _ in range(NBUF)],
    ]
    if E_up:
        scratch = [pltpu.VMEM((E_up,), jnp.int32)] + scratch

    def body(table_hbm, idx_hbm, *rest):
        if E_up:
            up_hbm, out_hbm, up_v, ia, ci, rows, isem, gsem, wsem = rest
            pltpu.sync_copy(up_hbm, up_v)
        else:
            out_hbm, ia, ci, rows, isem, gsem, wsem = rest
        wid = lax.axis_index("s") * 2 + lax.axis_index("c")
        base_c = wid * S

        for b in range(NBUF):
            pltpu.async_copy(idx_hbm.at[base_c + b], ia.at[b], isem[b])

        def rounds(t, _):
            for b in range(NBUF):
                s = t * NBUF + b
                c = base_c + s
                pltpu.make_async_copy(idx_hbm.at[c], ia.at[b], isem[b]).wait()
                if E_up:
                    for k in range(5):
                        for i in range(STEP_E // 16):
                            sl = pl.ds(i * 16, 16)
                            ci[b, k, sl] = plsc.load_gather(up_v, [ia[b, k, sl]])
                    idxbuf = ci
                else:
                    idxbuf = ia
                @pl.when(t > 0)
                def _drain():
                    pltpu.make_async_copy(
                        rows[b], out_hbm.at[:, c], wsem[b]).wait()
                for k in range(5):
                    pltpu.async_copy(table_hbm.at[idxbuf.at[b, k]],
                                     rows[b].at[k], gsem[b])
                pltpu.make_async_copy(out_hbm.at[:, c], rows[b], gsem[b]).wait()
                pltpu.async_copy(rows[b], out_hbm.at[:, c], wsem[b])
                pltpu.async_copy(idx_hbm.at[c + NBUF], ia.at[b], isem[b])
            return _

        lax.fori_loop(0, S // NBUF, rounds, 0)
        for b in range(NBUF):
            c_last = base_c + (S - NBUF) + b
            pltpu.make_async_copy(rows[b], out_hbm.at[:, c_last], wsem[b]).wait()
            pltpu.make_async_copy(idx_hbm.at[c_last], ia.at[b], isem[b]).wait()

    return pl.kernel(
        body,
        out_type=jax.ShapeDtypeStruct((5, n_chunks, STEP_E, C), jnp.float32),
        mesh=mesh,
        scratch_types=scratch,
        compiler_params=pltpu.CompilerParams(needs_layout_passes=False,
                                             use_tc_tiling_on_sc=False),
    )


def _tc_conv(E_pad, Cin, Cout, cat_C, relu, Eb=1024):
    nblk = E_pad // Eb
    nc = Eb // STEP_E
    assert nblk * Eb == E_pad

    def body(*refs):
        if cat_C:
            g_ref, w_ref, b_ref, d_ref, o_ref = refs
        else:
            g_ref, w_ref, b_ref, o_ref = refs
        f = [g_ref[k].reshape(Eb, Cin) for k in range(5)]
        acc = jnp.dot(f[0], w_ref[0], preferred_element_type=jnp.float32)
        acc += jnp.dot(f[1] + f[3], w_ref[1], preferred_element_type=jnp.float32)
        acc += jnp.dot(f[2] + f[4], w_ref[2], preferred_element_type=jnp.float32)
        acc += jnp.dot(jnp.abs(f[1] - f[3]), w_ref[3], preferred_element_type=jnp.float32)
        acc += jnp.dot(jnp.abs(f[2] - f[4]), w_ref[4], preferred_element_type=jnp.float32)
        acc += b_ref[0]
        if relu:
            acc = jnp.maximum(acc, 0.0)
        if cat_C:
            o_ref[...] = jnp.concatenate([acc, d_ref[...]], axis=-1)
        else:
            o_ref[...] = acc

    in_specs = [
        pl.BlockSpec((5, nc, STEP_E, Cin), lambda i: (0, i, 0, 0)),
        pl.BlockSpec((5, Cin, Cout), lambda i: (0, 0, 0)),
        pl.BlockSpec((1, Cout), lambda i: (0, 0)),
    ]
    if cat_C:
        in_specs.append(pl.BlockSpec((Eb, cat_C), lambda i: (i, 0)))

    return pl.pallas_call(
        body,
        grid=(nblk,),
        in_specs=in_specs,
        out_specs=pl.BlockSpec((Eb, Cout + cat_C), lambda i: (i, 0)),
        out_shape=jax.ShapeDtypeStruct((E_pad, Cout + cat_C), jnp.float32),
    )


def _prep_idx(g, E_pad):
    E = g.shape[0]
    gp = jnp.zeros((E_pad, 5), jnp.int32).at[:E].set(g.astype(jnp.int32))
    gp = gp.reshape(E_pad // STEP_E, STEP_E, 5).transpose(0, 2, 1)
    return jnp.concatenate(
        [gp, jnp.zeros((NBUF, 5, STEP_E), jnp.int32)], axis=0)


def _pad_rows(xT, E_pad):
    E, C = xT.shape
    return jnp.zeros((E_pad, C), xT.dtype).at[:E].set(xT)


def _pack_w(W):
    return jnp.transpose(W, (2, 1, 0)).astype(jnp.float32)


def kernel(fe, down0, down1, unpool_idx1, unpool_idx2, gemm1, gemm2,
           up0_W1, up0_b1, up0_W2, up0_b2, up1_W1, up1_b1, up1_W2, up1_b2):
    E0, E1, E2 = fe.shape[2], down0.shape[2], down1.shape[2]
    C0, C1, C2 = fe.shape[1], down0.shape[1], down1.shape[1]
    E1p = _ceil_to(E1, NW * STEP_E * NBUF)
    E2p = _ceil_to(E2, NW * STEP_E * NBUF)

    feT = fe[0].T
    d0T = _pad_rows(down0[0].T, E1p)
    d1T = _pad_rows(down1[0].T, E2p)
    idx1 = _prep_idx(gemm1[0], E1p)
    idx2 = _prep_idx(gemm2[0], E2p)
    up1 = unpool_idx1[0].astype(jnp.int32)
    up2 = unpool_idx2[0].astype(jnp.int32)
    W10, W20 = _pack_w(up0_W1), _pack_w(up0_W2)
    W11, W21 = _pack_w(up1_W1), _pack_w(up1_W2)
    b10, b20 = up0_b1.reshape(1, -1), up0_b2.reshape(1, -1)
    b11, b21 = up1_b1.reshape(1, -1), up1_b2.reshape(1, -1)

    G = _sc_gather(E0, C0, E1p, E1)(feT, idx1, up1)
    x = _tc_conv(E1p, C0, C1, C1, False)(G, W10, b10, d0T)
    G = _sc_gather(E1p, C0, E1p, 0)(x, idx1)
    x = _tc_conv(E1p, C0, C1, 0, True)(G, W10, b10)
    G = _sc_gather(E1p, C1, E1p, 0)(x, idx1)
    x = _tc_conv(E1p, C1, C1, 0, True)(G, W20, b20)

    G = _sc_gather(E1p, C1, E2p, E2)(x, idx2, up2)
    x = _tc_conv(E2p, C1, C2, C2, False)(G, W11, b11, d1T)
    G = _sc_gather(E2p, C1, E2p, 0)(x, idx2)
    x = _tc_conv(E2p, C1, C2, 0, True)(G, W11, b11)
    G = _sc_gather(E2p, C2, E2p, 0)(x, idx2)
    x = _tc_conv(E2p, C2, C2, 0, True)(G, W21, b21)

    return x[:E2].T[None]


# speedup vs baseline: 1.0176x; 1.0176x over previous
import functools

import jax
import jax.numpy as jnp
from jax import lax
from jax.experimental import pallas as pl
from jax.experimental.pallas import tpu as pltpu, tpu_sc as plsc

NW = 32
STEP_E = 32
NBUF = 2
FAST_FRAC = 0.70


def _ceil_to(x, m):
    return (x + m - 1) // m * m


def _sc_gather(E_in, C, E_pad, E_up):
    n_chunks = E_pad // STEP_E
    gran = 16 * NBUF
    c0 = int(round(n_chunks * FAST_FRAC / gran)) * gran
    c0 = min(max(c0, gran), n_chunks - gran)
    S_by_core = (c0 // 16, (n_chunks - c0) // 16)
    base_by_core = (0, c0)
    assert c0 % gran == 0 and (n_chunks - c0) % gran == 0

    mesh = plsc.VectorSubcoreMesh(core_axis_name="c", subcore_axis_name="s")
    scratch = [
        pltpu.VMEM((NBUF, 5, STEP_E), jnp.int32),
        pltpu.VMEM((NBUF, 5, STEP_E), jnp.int32),
        [pltpu.VMEM((5, STEP_E, C), jnp.float32) for _ in range(NBUF)],
        [pltpu.SemaphoreType.DMA for _ in range(NBUF)],
        [pltpu.SemaphoreType.DMA for _ in range(NBUF)],
        [pltpu.SemaphoreType.DMA for _ in range(NBUF)],
    ]
    if E_up:
        scratch = [pltpu.VMEM((E_up,), jnp.int32)] + scratch

    def body(table_hbm, idx_hbm, *rest):
        if E_up:
            up_hbm, out_hbm, up_v, ia, ci, rows, isem, gsem, wsem = rest
            pltpu.sync_copy(up_hbm, up_v)
        else:
            out_hbm, ia, ci, rows, isem, gsem, wsem = rest
        core = lax.axis_index("c")
        sub = lax.axis_index("s")
        S = jnp.where(core == 0, S_by_core[0], S_by_core[1])
        base_c = jnp.where(core == 0, base_by_core[0], base_by_core[1]) + sub * S

        for b in range(NBUF):
            pltpu.async_copy(idx_hbm.at[base_c + b], ia.at[b], isem[b])

        def rounds(t, _):
            for b in range(NBUF):
                s = t * NBUF + b
                c = base_c + s
                pltpu.make_async_copy(idx_hbm.at[c], ia.at[b], isem[b]).wait()
                if E_up:
                    for k in range(5):
                        for i in range(STEP_E // 16):
                            sl = pl.ds(i * 16, 16)
                            ci[b, k, sl] = plsc.load_gather(up_v, [ia[b, k, sl]])
                    idxbuf = ci
                else:
                    idxbuf = ia
                @pl.when(t > 0)
                def _drain():
                    pltpu.make_async_copy(
                        rows[b], out_hbm.at[:, c], wsem[b]).wait()
                for k in range(5):
                    pltpu.async_copy(table_hbm.at[idxbuf.at[b, k]],
                                     rows[b].at[k], gsem[b])
                pltpu.make_async_copy(out_hbm.at[:, c], rows[b], gsem[b]).wait()
                pltpu.async_copy(rows[b], out_hbm.at[:, c], wsem[b])
                pltpu.async_copy(idx_hbm.at[c + NBUF], ia.at[b], isem[b])
            return _

        lax.fori_loop(0, S // NBUF, rounds, 0)
        for b in range(NBUF):
            c_last = base_c + (S - NBUF) + b
            pltpu.make_async_copy(rows[b], out_hbm.at[:, c_last], wsem[b]).wait()
            pltpu.make_async_copy(idx_hbm.at[c_last], ia.at[b], isem[b]).wait()

    return pl.kernel(
        body,
        out_type=jax.ShapeDtypeStruct((5, n_chunks, STEP_E, C), jnp.float32),
        mesh=mesh,
        scratch_types=scratch,
        compiler_params=pltpu.CompilerParams(needs_layout_passes=False,
                                             use_tc_tiling_on_sc=False),
    )


def _tc_conv(E_pad, Cin, Cout, cat_C, relu, Eb=1024):
    nblk = E_pad // Eb
    nc = Eb // STEP_E
    assert nblk * Eb == E_pad

    def body(*refs):
        if cat_C:
            g_ref, w_ref, b_ref, d_ref, o_ref = refs
        else:
            g_ref, w_ref, b_ref, o_ref = refs
        f = [g_ref[k].reshape(Eb, Cin) for k in range(5)]
        acc = jnp.dot(f[0], w_ref[0], preferred_element_type=jnp.float32)
        acc += jnp.dot(f[1] + f[3], w_ref[1], preferred_element_type=jnp.float32)
        acc += jnp.dot(f[2] + f[4], w_ref[2], preferred_element_type=jnp.float32)
        acc += jnp.dot(jnp.abs(f[1] - f[3]), w_ref[3], preferred_element_type=jnp.float32)
        acc += jnp.dot(jnp.abs(f[2] - f[4]), w_ref[4], preferred_element_type=jnp.float32)
        acc += b_ref[0]
        if relu:
            acc = jnp.maximum(acc, 0.0)
        if cat_C:
            o_ref[...] = jnp.concatenate([acc, d_ref[...]], axis=-1)
        else:
            o_ref[...] = acc

    in_specs = [
        pl.BlockSpec((5, nc, STEP_E, Cin), lambda i: (0, i, 0, 0)),
        pl.BlockSpec((5, Cin, Cout), lambda i: (0, 0, 0)),
        pl.BlockSpec((1, Cout), lambda i: (0, 0)),
    ]
    if cat_C:
        in_specs.append(pl.BlockSpec((Eb, cat_C), lambda i: (i, 0)))

    return pl.pallas_call(
        body,
        grid=(nblk,),
        in_specs=in_specs,
        out_specs=pl.BlockSpec((Eb, Cout + cat_C), lambda i: (i, 0)),
        out_shape=jax.ShapeDtypeStruct((E_pad, Cout + cat_C), jnp.float32),
    )


def _prep_idx(g, E_pad):
    E = g.shape[0]
    gp = jnp.zeros((E_pad, 5), jnp.int32).at[:E].set(g.astype(jnp.int32))
    gp = gp.reshape(E_pad // STEP_E, STEP_E, 5).transpose(0, 2, 1)
    return jnp.concatenate(
        [gp, jnp.zeros((NBUF, 5, STEP_E), jnp.int32)], axis=0)


def _pad_rows(xT, E_pad):
    E, C = xT.shape
    return jnp.zeros((E_pad, C), xT.dtype).at[:E].set(xT)


def _pack_w(W):
    return jnp.transpose(W, (2, 1, 0)).astype(jnp.float32)


def kernel(fe, down0, down1, unpool_idx1, unpool_idx2, gemm1, gemm2,
           up0_W1, up0_b1, up0_W2, up0_b2, up1_W1, up1_b1, up1_W2, up1_b2):
    E0, E1, E2 = fe.shape[2], down0.shape[2], down1.shape[2]
    C0, C1, C2 = fe.shape[1], down0.shape[1], down1.shape[1]
    E1p = _ceil_to(E1, NW * STEP_E * NBUF)
    E2p = _ceil_to(E2, NW * STEP_E * NBUF)

    feT = fe[0].T
    d0T = _pad_rows(down0[0].T, E1p)
    d1T = _pad_rows(down1[0].T, E2p)
    idx1 = _prep_idx(gemm1[0], E1p)
    idx2 = _prep_idx(gemm2[0], E2p)
    up1 = unpool_idx1[0].astype(jnp.int32)
    up2 = unpool_idx2[0].astype(jnp.int32)
    W10, W20 = _pack_w(up0_W1), _pack_w(up0_W2)
    W11, W21 = _pack_w(up1_W1), _pack_w(up1_W2)
    b10, b20 = up0_b1.reshape(1, -1), up0_b2.reshape(1, -1)
    b11, b21 = up1_b1.reshape(1, -1), up1_b2.reshape(1, -1)

    G = _sc_gather(E0, C0, E1p, E1)(feT, idx1, up1)
    x = _tc_conv(E1p, C0, C1, C1, False)(G, W10, b10, d0T)
    G = _sc_gather(E1p, C0, E1p, 0)(x, idx1)
    x = _tc_conv(E1p, C0, C1, 0, True)(G, W10, b10)
    G = _sc_gather(E1p, C1, E1p, 0)(x, idx1)
    x = _tc_conv(E1p, C1, C1, 0, True)(G, W20, b20)

    G = _sc_gather(E1p, C1, E2p, E2)(x, idx2, up2)
    x = _tc_conv(E2p, C1, C2, C2, False)(G, W11, b11, d1T)
    G = _sc_gather(E2p, C1, E2p, 0)(x, idx2)
    x = _tc_conv(E2p, C1, C2, 0, True)(G, W11, b11)
    G = _sc_gather(E2p, C2, E2p, 0)(x, idx2)
    x = _tc_conv(E2p, C2, C2, 0, True)(G, W21, b21)

    return x[:E2].T[None]


# speedup vs baseline: 1.8110x; 1.7796x over previous
import functools

import jax
import jax.numpy as jnp
from jax import lax
from jax.experimental import pallas as pl
from jax.experimental.pallas import tpu as pltpu, tpu_sc as plsc

NW = 32
STEP_E = 32
FAST_FRAC = 0.78


def _ceil_to(x, m):
    return (x + m - 1) // m * m


def _sc_gather(C, E, E_pad, compose, nbuf):
    n_chunks = E_pad // STEP_E
    n_valid = E // STEP_E
    gran = 16 * nbuf
    c0 = int(round(n_chunks * FAST_FRAC / gran)) * gran
    c0 = min(max(c0, gran), n_chunks - gran)
    S_by_core = (c0 // 16, (n_chunks - c0) // 16)
    assert S_by_core[1] * 16 + c0 == n_chunks and (n_chunks - c0) % gran == 0

    mesh = plsc.VectorSubcoreMesh(core_axis_name="c", subcore_axis_name="s")
    scratch = [
        pltpu.VMEM((nbuf, STEP_E, 5), jnp.int32),
        pltpu.VMEM((nbuf, 5, STEP_E), jnp.int32),
        [pltpu.VMEM((5, STEP_E, C), jnp.float32) for _ in range(nbuf)],
        [pltpu.SemaphoreType.DMA for _ in range(nbuf)],
        [pltpu.SemaphoreType.DMA for _ in range(nbuf)],
        [pltpu.SemaphoreType.DMA for _ in range(nbuf)],
    ]
    if compose:
        scratch = [pltpu.VMEM((compose,), jnp.int32)] + scratch

    def body(table_hbm, gemm_hbm, *rest):
        if compose:
            up_hbm, out_hbm, up_v, ib, ci, rows, isem, gsem, wsem = rest
            pltpu.sync_copy(up_hbm, up_v)
        else:
            out_hbm, ib, ci, rows, isem, gsem, wsem = rest
        core = lax.axis_index("c")
        sub = lax.axis_index("s")
        S = jnp.where(core == 0, S_by_core[0], S_by_core[1])
        base_c = jnp.where(core == 0, 0, c0) + sub * S

        def idx_src(c):
            ce = jnp.minimum(c, n_valid - 1)
            return gemm_hbm.at[pl.ds(ce * STEP_E, STEP_E), :]

        for b in range(nbuf):
            pltpu.async_copy(idx_src(base_c + b), ib.at[b], isem[b])

        lanes = jnp.arange(16, dtype=jnp.int32)

        def rounds(t, _):
            for b in range(nbuf):
                s = t * nbuf + b
                c = base_c + s
                pltpu.make_async_copy(idx_src(c), ib.at[b], isem[b]).wait()
                for k in range(5):
                    for i in range(2):
                        row = lanes + (16 * i)
                        col = jnp.full((16,), k, jnp.int32)
                        v = plsc.load_gather(ib.at[b], [row, col])
                        if compose:
                            v = plsc.load_gather(up_v, [v])
                        ci[b, k, pl.ds(16 * i, 16)] = v
                @pl.when(t > 0)
                def _drain():
                    pltpu.make_async_copy(
                        rows[b], out_hbm.at[:, c], wsem[b]).wait()
                for k in range(5):
                    pltpu.async_copy(table_hbm.at[ci.at[b, k]],
                                     rows[b].at[k], gsem[b])
                pltpu.make_async_copy(out_hbm.at[:, c], rows[b], gsem[b]).wait()
                pltpu.async_copy(rows[b], out_hbm.at[:, c], wsem[b])
                pltpu.async_copy(idx_src(c + nbuf), ib.at[b], isem[b])
            return _

        lax.fori_loop(0, S // nbuf, rounds, 0)
        for b in range(nbuf):
            c_last = base_c + (S - nbuf) + b
            pltpu.make_async_copy(rows[b], out_hbm.at[:, c_last], wsem[b]).wait()
            pltpu.make_async_copy(idx_src(c_last), ib.at[b], isem[b]).wait()

    return pl.kernel(
        body,
        out_type=jax.ShapeDtypeStruct((5, n_chunks, STEP_E, C), jnp.float32),
        mesh=mesh,
        scratch_types=scratch,
        compiler_params=pltpu.CompilerParams(needs_layout_passes=False,
                                             use_tc_tiling_on_sc=False),
    )


def _tc_conv(E_pad, Cin, Cout, cat_C, relu, out_T=False, Eb=1024):
    nblk = E_pad // Eb
    nc = Eb // STEP_E
    assert nblk * Eb == E_pad

    def body(*refs):
        if cat_C:
            g_ref, w_ref, b_ref, d_ref, o_ref = refs
        else:
            g_ref, w_ref, b_ref, o_ref = refs
        f = [g_ref[k].reshape(Eb, Cin) for k in range(5)]
        s = [f[0], f[1] + f[3], f[2] + f[4],
             jnp.abs(f[1] - f[3]), jnp.abs(f[2] - f[4])]
        if out_T:
            acc = None
            for k in range(5):
                t = lax.dot_general(w_ref[k], s[k], (((0,), (1,)), ((), ())),
                                    preferred_element_type=jnp.float32)
                acc = t if acc is None else acc + t
            acc += b_ref[...]
            if relu:
                acc = jnp.maximum(acc, 0.0)
            o_ref[...] = acc
            return
        acc = None
        for k in range(5):
            t = jnp.dot(s[k], w_ref[k], preferred_element_type=jnp.float32)
            acc = t if acc is None else acc + t
        acc += b_ref[0]
        if relu:
            acc = jnp.maximum(acc, 0.0)
        if cat_C:
            o_ref[...] = jnp.concatenate([acc, d_ref[...]], axis=-1)
        else:
            o_ref[...] = acc

    in_specs = [
        pl.BlockSpec((5, nc, STEP_E, Cin), lambda i: (0, i, 0, 0)),
        pl.BlockSpec((5, Cin, Cout), lambda i: (0, 0, 0)),
        pl.BlockSpec((Cout, 1) if out_T else (1, Cout), lambda i: (0, 0)),
    ]
    if cat_C:
        in_specs.append(pl.BlockSpec((Eb, cat_C), lambda i: (i, 0)))
    if out_T:
        out_specs = pl.BlockSpec((Cout, Eb), lambda i: (0, i))
        out_shape = jax.ShapeDtypeStruct((Cout, E_pad), jnp.float32)
    else:
        out_specs = pl.BlockSpec((Eb, Cout + cat_C), lambda i: (i, 0))
        out_shape = jax.ShapeDtypeStruct((E_pad, Cout + cat_C), jnp.float32)

    return pl.pallas_call(
        body,
        grid=(nblk,),
        in_specs=in_specs,
        out_specs=out_specs,
        out_shape=out_shape,
    )


def _pad_rows(xT, E_pad):
    E, C = xT.shape
    return jnp.zeros((E_pad, C), xT.dtype).at[:E].set(xT)


def _pack_w(W):
    return jnp.transpose(W, (2, 1, 0)).astype(jnp.float32)


def kernel(fe, down0, down1, unpool_idx1, unpool_idx2, gemm1, gemm2,
           up0_W1, up0_b1, up0_W2, up0_b2, up1_W1, up1_b1, up1_W2, up1_b2):
    E0, E1, E2 = fe.shape[2], down0.shape[2], down1.shape[2]
    C0, C1, C2 = fe.shape[1], down0.shape[1], down1.shape[1]
    E1p = _ceil_to(E1, 1024)
    E2p = _ceil_to(E2, 1024)

    feT = fe[0].T
    d0T = _pad_rows(down0[0].T, E1p)
    d1T = _pad_rows(down1[0].T, E2p)
    g1 = gemm1[0].astype(jnp.int32)
    g2 = gemm2[0].astype(jnp.int32)
    up1 = unpool_idx1[0].astype(jnp.int32)
    up2 = unpool_idx2[0].astype(jnp.int32)
    W10, W20 = _pack_w(up0_W1), _pack_w(up0_W2)
    W11, W21 = _pack_w(up1_W1), _pack_w(up1_W2)
    b10, b20 = up0_b1.reshape(1, -1), up0_b2.reshape(1, -1)
    b11 = up1_b1.reshape(1, -1)
    b21 = up1_b2.reshape(-1, 1)

    G = _sc_gather(C0, E1, E1p, E1, 2)(feT, g1, up1)
    x = _tc_conv(E1p, C0, C1, C1, False)(G, W10, b10, d0T)
    G = _sc_gather(C0, E1, E1p, 0, 2)(x, g1)
    x = _tc_conv(E1p, C0, C1, 0, True)(G, W10, b10)
    G = _sc_gather(C1, E1, E1p, 0, 2)(x, g1)
    x = _tc_conv(E1p, C1, C1, 0, True)(G, W20, b20)

    G = _sc_gather(C1, E2, E2p, E2, 2)(x, g2, up2)
    x = _tc_conv(E2p, C1, C2, C2, False)(G, W11, b11, d1T)
    G = _sc_gather(C1, E2, E2p, 0, 2)(x, g2)
    x = _tc_conv(E2p, C1, C2, 0, True)(G, W11, b11)
    G = _sc_gather(C2, E2, E2p, 0, 2)(x, g2)
    x = _tc_conv(E2p, C2, C2, 0, True, out_T=True)(G, W21, b21)

    return x[:, :E2][None]


# speedup vs baseline: 1.9454x; 1.0742x over previous
import functools

import jax
import jax.numpy as jnp
from jax import lax
from jax.experimental import pallas as pl
from jax.experimental.pallas import tpu as pltpu, tpu_sc as plsc

NW = 32
STEP_E = 32
FAST_FRAC = 0.60


def _ceil_to(x, m):
    return (x + m - 1) // m * m


def _sc_gather(C, E, E_pad, compose, nbuf):
    n_chunks = E_pad // STEP_E
    n_valid = E // STEP_E
    gran = 16 * nbuf
    c0 = int(round(n_chunks * FAST_FRAC / gran)) * gran
    c0 = min(max(c0, gran), n_chunks - gran)
    S_by_core = (c0 // 16, (n_chunks - c0) // 16)
    assert S_by_core[1] * 16 + c0 == n_chunks and (n_chunks - c0) % gran == 0

    mesh = plsc.VectorSubcoreMesh(core_axis_name="c", subcore_axis_name="s")
    scratch = [
        pltpu.VMEM((nbuf, STEP_E, 5), jnp.int32),
        pltpu.VMEM((nbuf, 5, STEP_E), jnp.int32),
        [pltpu.VMEM((5, STEP_E, C), jnp.float32) for _ in range(nbuf)],
        [pltpu.SemaphoreType.DMA for _ in range(nbuf)],
        [pltpu.SemaphoreType.DMA for _ in range(nbuf)],
        [pltpu.SemaphoreType.DMA for _ in range(nbuf)],
    ]
    if compose:
        scratch = [pltpu.VMEM((compose,), jnp.int32)] + scratch

    def body(table_hbm, gemm_hbm, *rest):
        if compose:
            up_hbm, out_hbm, up_v, ib, ci, rows, isem, gsem, wsem = rest
            pltpu.sync_copy(up_hbm, up_v)
        else:
            out_hbm, ib, ci, rows, isem, gsem, wsem = rest
        core = lax.axis_index("c")
        sub = lax.axis_index("s")
        S = jnp.where(core == 0, S_by_core[0], S_by_core[1])
        base_c = jnp.where(core == 0, 0, c0) + sub * S

        def idx_src(c):
            ce = jnp.minimum(c, n_valid - 1)
            return gemm_hbm.at[pl.ds(ce * STEP_E, STEP_E), :]

        for b in range(nbuf):
            pltpu.async_copy(idx_src(base_c + b), ib.at[b], isem[b])

        lanes = jnp.arange(16, dtype=jnp.int32)

        def rounds(t, _):
            for b in range(nbuf):
                s = t * nbuf + b
                c = base_c + s
                pltpu.make_async_copy(idx_src(c), ib.at[b], isem[b]).wait()
                for k in range(5):
                    for i in range(2):
                        row = lanes + (16 * i)
                        col = jnp.full((16,), k, jnp.int32)
                        v = plsc.load_gather(ib.at[b], [row, col])
                        if compose:
                            v = plsc.load_gather(up_v, [v])
                        ci[b, k, pl.ds(16 * i, 16)] = v
                @pl.when(t > 0)
                def _drain():
                    pltpu.make_async_copy(
                        rows[b], out_hbm.at[:, c], wsem[b]).wait()
                for k in range(5):
                    pltpu.async_copy(table_hbm.at[ci.at[b, k]],
                                     rows[b].at[k], gsem[b])
                pltpu.make_async_copy(out_hbm.at[:, c], rows[b], gsem[b]).wait()
                pltpu.async_copy(rows[b], out_hbm.at[:, c], wsem[b])
                pltpu.async_copy(idx_src(c + nbuf), ib.at[b], isem[b])
            return _

        lax.fori_loop(0, S // nbuf, rounds, 0)
        for b in range(nbuf):
            c_last = base_c + (S - nbuf) + b
            pltpu.make_async_copy(rows[b], out_hbm.at[:, c_last], wsem[b]).wait()
            pltpu.make_async_copy(idx_src(c_last), ib.at[b], isem[b]).wait()

    return pl.kernel(
        body,
        out_type=jax.ShapeDtypeStruct((5, n_chunks, STEP_E, C), jnp.float32),
        mesh=mesh,
        scratch_types=scratch,
        compiler_params=pltpu.CompilerParams(needs_layout_passes=False,
                                             use_tc_tiling_on_sc=False),
    )


def _tc_conv(E_pad, Cin, Cout, cat_C, relu, out_T=False, E_out=None, Eb=1024):
    nblk = _ceil_to(E_out, Eb) // Eb if out_T else E_pad // Eb
    nc = Eb // STEP_E

    def body(*refs):
        if cat_C:
            g_ref, w_ref, b_ref, d_ref, o_ref = refs
        else:
            g_ref, w_ref, b_ref, o_ref = refs
        f = [g_ref[k].reshape(Eb, Cin) for k in range(5)]
        s = [f[0], f[1] + f[3], f[2] + f[4],
             jnp.abs(f[1] - f[3]), jnp.abs(f[2] - f[4])]
        if out_T:
            acc = None
            for k in range(5):
                t = lax.dot_general(w_ref[k], s[k], (((0,), (1,)), ((), ())),
                                    preferred_element_type=jnp.float32)
                acc = t if acc is None else acc + t
            acc += b_ref[...]
            if relu:
                acc = jnp.maximum(acc, 0.0)
            o_ref[...] = acc
            return
        acc = None
        for k in range(5):
            t = jnp.dot(s[k], w_ref[k], preferred_element_type=jnp.float32)
            acc = t if acc is None else acc + t
        acc += b_ref[0]
        if relu:
            acc = jnp.maximum(acc, 0.0)
        if cat_C:
            o_ref[...] = jnp.concatenate([acc, d_ref[...]], axis=-1)
        else:
            o_ref[...] = acc

    in_specs = [
        pl.BlockSpec((5, nc, STEP_E, Cin), lambda i: (0, i, 0, 0)),
        pl.BlockSpec((5, Cin, Cout), lambda i: (0, 0, 0)),
        pl.BlockSpec((Cout, 1) if out_T else (1, Cout), lambda i: (0, 0)),
    ]
    if cat_C:
        in_specs.append(pl.BlockSpec((Eb, cat_C), lambda i: (i, 0)))
    if out_T:
        out_specs = pl.BlockSpec((Cout, Eb), lambda i: (0, i))
        out_shape = jax.ShapeDtypeStruct((Cout, E_out), jnp.float32)
    else:
        out_specs = pl.BlockSpec((Eb, Cout + cat_C), lambda i: (i, 0))
        out_shape = jax.ShapeDtypeStruct((E_pad, Cout + cat_C), jnp.float32)

    return pl.pallas_call(
        body,
        grid=(nblk,),
        in_specs=in_specs,
        out_specs=out_specs,
        out_shape=out_shape,
    )


def _tc_transpose(C, E, E_pad, Eb=1024):
    nblk = _ceil_to(E_pad, Eb) // Eb

    def body(x_ref, o_ref):
        o_ref[...] = x_ref[...].T

    return pl.pallas_call(
        body,
        grid=(nblk,),
        in_specs=[pl.BlockSpec((C, Eb), lambda i: (0, i))],
        out_specs=pl.BlockSpec((Eb, C), lambda i: (i, 0)),
        out_shape=jax.ShapeDtypeStruct((E_pad, C), jnp.float32),
    )


def _pack_w(W):
    return jnp.transpose(W, (2, 1, 0)).astype(jnp.float32)


def kernel(fe, down0, down1, unpool_idx1, unpool_idx2, gemm1, gemm2,
           up0_W1, up0_b1, up0_W2, up0_b2, up1_W1, up1_b1, up1_W2, up1_b2):
    E0, E1, E2 = fe.shape[2], down0.shape[2], down1.shape[2]
    C0, C1, C2 = fe.shape[1], down0.shape[1], down1.shape[1]
    E1p = _ceil_to(E1, 1024)
    E2p = _ceil_to(E2, 1024)

    feT = _tc_transpose(C0, E0, E0)(fe[0])
    d0T = _tc_transpose(C1, E1, E1p)(down0[0])
    d1T = _tc_transpose(C2, E2, E2p)(down1[0])
    g1 = gemm1[0].astype(jnp.int32)
    g2 = gemm2[0].astype(jnp.int32)
    up1 = unpool_idx1[0].astype(jnp.int32)
    up2 = unpool_idx2[0].astype(jnp.int32)
    W10, W20 = _pack_w(up0_W1), _pack_w(up0_W2)
    W11, W21 = _pack_w(up1_W1), _pack_w(up1_W2)
    b10, b20 = up0_b1.reshape(1, -1), up0_b2.reshape(1, -1)
    b11 = up1_b1.reshape(1, -1)
    b21 = up1_b2.reshape(-1, 1)

    G = _sc_gather(C0, E1, E1p, E1, 2)(feT, g1, up1)
    x = _tc_conv(E1p, C0, C1, C1, False)(G, W10, b10, d0T)
    G = _sc_gather(C0, E1, E1p, 0, 2)(x, g1)
    x = _tc_conv(E1p, C0, C1, 0, True)(G, W10, b10)
    G = _sc_gather(C1, E1, E1p, 0, 2)(x, g1)
    x = _tc_conv(E1p, C1, C1, 0, True)(G, W20, b20)

    G = _sc_gather(C1, E2, E2p, E2, 2)(x, g2, up2)
    x = _tc_conv(E2p, C1, C2, C2, False)(G, W11, b11, d1T)
    G = _sc_gather(C1, E2, E2p, 0, 2)(x, g2)
    x = _tc_conv(E2p, C1, C2, 0, True)(G, W11, b11)
    G = _sc_gather(C2, E2, E2p, 0, 2)(x, g2)
    x = _tc_conv(E2p, C2, C2, 0, True, out_T=True, E_out=E2)(G, W21, b21)

    return x[None]
